# baseline (speedup 1.0000x reference)
import math

import jax
import jax.numpy as jnp
from jax import lax
from jax.experimental import pallas as pl

N = 192
M = N - 1
P = N * M
NW = 8
NC = 32
CB = 8
PB = CB * M
NBLK = N // CB
C0 = 0.28209479177387814
C1 = 0.4886025119029199
F32 = jnp.float32
_INTERP = False


def _silu(x):
    return x * jax.nn.sigmoid(x)


def _seg_matrix():
    ia = lax.broadcasted_iota(jnp.int32, (CB, PB), 0)
    ir = lax.broadcasted_iota(jnp.int32, (CB, PB), 1)
    lo = ia * M
    return jnp.where((ir >= lo) & (ir < lo + M), F32(1.0), F32(0.0))


def _onehot_pairs(idx_col):
    it = lax.broadcasted_iota(jnp.int32, (PB, N), 1)
    return jnp.where(idx_col == it, F32(1.0), F32(0.0))


def _pair_kernel(ids, emb, cart, cc0,
                 nw1, nb1, nw2, nb2, nw3, nb3,
                 cw1, cb1, cw2, cb2, cw3, cb3,
                 co_ref, meta_ref):
    dot = lambda a, b: jnp.dot(a, b, preferred_element_type=F32)
    x = emb[...]
    h = _silu(dot(x, nw1[...]) + nb1[...])
    h = _silu(dot(h, nw2[...]) + nb2[...])
    e = dot(h, nw3[...]) + nb3[...]

    idsv = ids[...]
    ohc = _onehot_pairs(idsv[:, 0:1])
    ohn = _onehot_pairs(idsv[:, 1:2])
    cartv = cart[...]
    dv = dot(ohn, cartv) - dot(ohc, cartv)
    d2 = jnp.sum(dv * dv, axis=1, keepdims=True)
    dist = jnp.sqrt(d2)

    w = e[:, 0:NW]
    cen = e[:, NW:2 * NW]
    t = w * (dist - cen)
    rad8 = jnp.exp(-(t * t))
    r0 = dot(rad8, cc0[...])

    u = dv / dist
    s1v = C1 * u[:, 1:2]
    s2v = C1 * u[:, 2:3]
    s3v = C1 * u[:, 0:1]

    wr0 = e[:, 2 * NW:] * r0
    lo, hi = wr0[:, 0:NC], wr0[:, NC:]
    worb = jnp.concatenate(
        [lo * C0, hi * s1v, hi * s2v, hi * s3v], axis=1)

    h2 = _silu(dot(x, cw1[...]) + cb1[...])
    h2 = _silu(dot(h2, cw2[...]) + cb2[...])
    cg = dot(h2, cw3[...]) + cb3[...]

    meta_ref[...] = jnp.concatenate(
        [rad8, jnp.full((PB, 1), C0, dtype=F32), s1v, s2v, s3v, cg], axis=1)
    co_ref[...] = dot(_seg_matrix(), worb) * F32(1.0 / M)


def _inter_kernel(ids, meta, cc, wmat, co_tab, cf_tab, co_ref, *, it):
    dot = lambda a, b: jnp.dot(a, b, preferred_element_type=F32)
    m = meta[...]
    rad8 = m[:, 0:NW]
    cg = m[:, 12 + 12 * it:24 + 12 * it]

    ohn = _onehot_pairs(ids[...][:, 1:2])
    g = dot(ohn, co_tab[...])
    cf = dot(ohn, cf_tab[...])

    r = dot(rad8, cc[...])
    wr = r * cf
    lo, hi = wr[:, 0:NC], wr[:, NC:]
    wl = [lo, hi, hi, hi]
    orbs = [wl[s] * m[:, NW + s:NW + s + 1] for s in range(4)]
    gs = [g[:, s * NC:(s + 1) * NC] for s in range(4)]

    wgt = dot(cg, wmat[...])
    acc = [None] * 4
    for a in range(4):
        for b in range(4):
            prod = gs[a] * orbs[b]
            for s in range(4):
                col = wgt[:, (a * 4 + b) * 4 + s:(a * 4 + b) * 4 + s + 1]
                term = col * prod
                acc[s] = term if acc[s] is None else acc[s] + term
    wo = jnp.concatenate(acc, axis=1)
    co_ref[...] = dot(_seg_matrix(), wo) * F32(1.0 / M)


def _density_add(co):
    sq = co * co
    return jnp.concatenate(
        [sq[:, 0:NC], sq[:, NC:2 * NC] + sq[:, 2 * NC:3 * NC] + sq[:, 3 * NC:]],
        axis=1)


def _atom_kernel(co, dprev, w1, b1, w2, b2, w3, b3, dens_ref, coeff_ref, *, scale):
    dot = lambda a, b: jnp.dot(a, b, preferred_element_type=F32)
    dens = (dprev[...] + _density_add(co[...])) * F32(scale)
    dens_ref[...] = dens
    h = _silu(dot(dens, w1[...]) + b1[...])
    h = _silu(dot(h, w2[...]) + b2[...])
    coeff_ref[...] = dot(h, w3[...]) + b3[...]


def _final_kernel(co, dprev, w1, b1, w2, b2, w3, b3, out_ref):
    dot = lambda a, b: jnp.dot(a, b, preferred_element_type=F32)
    dens = (dprev[...] + _density_add(co[...])) * F32(1.0 / math.sqrt(2.0))
    h = _silu(dot(dens, w1[...]) + b1[...])
    h = _silu(dot(h, w2[...]) + b2[...])
    y = dot(h, w3[...]) + b3[...]
    out_ref[...] = jnp.sum(y, axis=0, keepdims=True)


def _full(shape):
    nd = len(shape)
    return pl.BlockSpec(shape, lambda i: (0,) * nd)


def kernel(cart, input_embed, contracted_coeff, ens_cg, p_neigh, p_cg, p_mp, p_out,
           neighlist, index_l, index_i1, index_i2, index_add, index_cg):
    f32 = lambda a: a.astype(F32)
    ids = neighlist.T.astype(jnp.int32)
    cc = f32(contracted_coeff).reshape(3, 2 * NC, NW).transpose(0, 2, 1)

    def mlp_args(p):
        w1, b1, w2, b2, w3, b3 = p
        return [f32(w1), f32(b1).reshape(1, -1), f32(w2), f32(b2).reshape(1, -1),
                f32(w3), f32(b3).reshape(1, -1)]

    oh1 = jax.nn.one_hot(index_i1, 4, dtype=F32)
    oh2 = jax.nn.one_hot(index_i2, 4, dtype=F32)
    ohadd = jax.nn.one_hot(index_add, 4, dtype=F32)
    wmat = jnp.einsum('c,ca,cb,cs->cabs', f32(ens_cg), oh1, oh2, ohadd).reshape(12, 64)

    pair_specs = (
        [pl.BlockSpec((PB, 2), lambda i: (i, 0)),
         pl.BlockSpec((PB, 16), lambda i: (i, 0)),
         _full((N, 3)), _full((NW, 64))]
        + [_full(s.shape) for s in mlp_args(p_neigh)]
        + [_full(s.shape) for s in mlp_args(p_cg)]
    )
    co0, meta = pl.pallas_call(
        _pair_kernel,
        grid=(NBLK,),
        in_specs=pair_specs,
        out_specs=[pl.BlockSpec((CB, 128), lambda i: (i, 0)),
                   pl.BlockSpec((PB, 36), lambda i: (i, 0))],
        out_shape=[jax.ShapeDtypeStruct((N, 128), F32),
                   jax.ShapeDtypeStruct((P, 36), F32)],
        interpret=_INTERP,
    )(ids, f32(input_embed), f32(cart), cc[0],
      *mlp_args(p_neigh), *mlp_args(p_cg))

    def atom_step(co, dprev, params, scale):
        fn = lambda *refs: _atom_kernel(*refs, scale=scale)
        return pl.pallas_call(
            fn,
            grid=(1,),
            in_specs=[_full((N, 128)), _full((N, 64))] + [_full(s.shape) for s in mlp_args(params)],
            out_specs=[_full((N, 64)), _full((N, 64))],
            out_shape=[jax.ShapeDtypeStruct((N, 64), F32),
                       jax.ShapeDtypeStruct((N, 64), F32)],
            interpret=_INTERP,
        )(co, dprev, *mlp_args(params))

    def inter_step(it, co_tab, cf_tab):
        fn = lambda *refs: _inter_kernel(*refs, it=it)
        return pl.pallas_call(
            fn,
            grid=(NBLK,),
            in_specs=[pl.BlockSpec((PB, 2), lambda i: (i, 0)),
                      pl.BlockSpec((PB, 36), lambda i: (i, 0)),
                      _full((NW, 64)), _full((12, 64)),
                      _full((N, 128)), _full((N, 64))],
            out_specs=pl.BlockSpec((CB, 128), lambda i: (i, 0)),
            out_shape=jax.ShapeDtypeStruct((N, 128), F32),
            interpret=_INTERP,
        )(ids, meta, cc[it + 1], wmat, co_tab, cf_tab)

    zeros64 = jnp.zeros((N, 64), dtype=F32)
    dens0, coeff0 = atom_step(co0, zeros64, p_mp[0], 1.0)
    co1 = inter_step(0, co0, coeff0)
    dens1, coeff1 = atom_step(co1, dens0, p_mp[1], 1.0 / math.sqrt(2.0))
    co2 = inter_step(1, co1, coeff1)

    out = pl.pallas_call(
        _final_kernel,
        grid=(1,),
        in_specs=[_full((N, 128)), _full((N, 64))] + [_full(s.shape) for s in mlp_args(p_out)],
        out_specs=_full((1, 1)),
        out_shape=jax.ShapeDtypeStruct((1, 1), F32),
        interpret=_INTERP,
    )(co2, dens1, *mlp_args(p_out))
    return out[0, 0]


# speedup vs baseline: 3.0214x; 3.0214x over previous
import math

import jax
import jax.numpy as jnp
from jax import lax
from jax.experimental import pallas as pl

N = 192
M = N - 1
P = N * M
NW = 8
NC = 32
CB = 8
PB = CB * M
NBLK = N // CB
C0 = 0.28209479177387814
C1 = 0.4886025119029199
F32 = jnp.float32
_INTERP = False


def _silu(x):
    return x * jax.nn.sigmoid(x)


def _seg_matrix():
    ia = lax.broadcasted_iota(jnp.int32, (CB, PB), 0)
    ir = lax.broadcasted_iota(jnp.int32, (CB, PB), 1)
    lo = ia * M
    return jnp.where((ir >= lo) & (ir < lo + M), F32(1.0), F32(0.0))


def _onehot_pairs(idx_col):
    it = lax.broadcasted_iota(jnp.int32, (PB, N), 1)
    return jnp.where(idx_col == it, F32(1.0), F32(0.0))


def _pair_kernel(ids, emb, cart, cc0,
                 nw1, nb1, nw2, nb2, nw3, nb3,
                 cw1, cb1, cw2, cb2, cw3, cb3,
                 co_ref, meta_ref):
    dot = lambda a, b: jnp.dot(a, b, preferred_element_type=F32)
    x = emb[...]
    h = _silu(dot(x, nw1[...]) + nb1[...])
    h = _silu(dot(h, nw2[...]) + nb2[...])
    e = dot(h, nw3[...]) + nb3[...]

    idsv = ids[...]
    ohc = _onehot_pairs(idsv[:, 0:1])
    ohn = _onehot_pairs(idsv[:, 1:2])
    cartv = cart[...]
    dv = dot(ohn, cartv) - dot(ohc, cartv)
    d2 = jnp.sum(dv * dv, axis=1, keepdims=True)
    dist = jnp.sqrt(d2)

    w = e[:, 0:NW]
    cen = e[:, NW:2 * NW]
    t = w * (dist - cen)
    rad8 = jnp.exp(-(t * t))
    r0 = dot(rad8, cc0[...])

    u = dv / dist
    s1v = C1 * u[:, 1:2]
    s2v = C1 * u[:, 2:3]
    s3v = C1 * u[:, 0:1]

    wr0 = e[:, 2 * NW:] * r0
    lo, hi = wr0[:, 0:NC], wr0[:, NC:]
    worb = jnp.concatenate(
        [lo * C0, hi * s1v, hi * s2v, hi * s3v], axis=1)

    h2 = _silu(dot(x, cw1[...]) + cb1[...])
    h2 = _silu(dot(h2, cw2[...]) + cb2[...])
    cg = dot(h2, cw3[...]) + cb3[...]

    meta_ref[...] = jnp.concatenate(
        [rad8, jnp.full((PB, 1), C0, dtype=F32), s1v, s2v, s3v, cg], axis=1)
    co_ref[...] = dot(_seg_matrix(), worb) * F32(1.0 / M)


def _inter_kernel(ids, meta, cc, wmatw, sphw, co_tab, cf_tab, co_ref, *, it):
    dot = lambda a, b: jnp.dot(a, b, preferred_element_type=F32)
    m = meta[...]
    rad8 = m[:, 0:NW]
    sph4 = m[:, NW:NW + 4]
    cg = m[:, 12 + 12 * it:24 + 12 * it]

    ohn = _onehot_pairs(ids[...][:, 1:2])
    g = dot(ohn, co_tab[...])
    cf = dot(ohn, cf_tab[...])

    r = dot(rad8, cc[...])
    wr = r * cf
    hi = wr[:, NC:]
    orb = jnp.concatenate([wr[:, 0:NC], hi, hi, hi], axis=1) * dot(sph4, sphw[...])
    orbs = [orb[:, s * NC:(s + 1) * NC] for s in range(4)]
    gs = [g[:, s * NC:(s + 1) * NC] for s in range(4)]

    wgtb = dot(cg, wmatw[...])
    acc = [None] * 4
    for a in range(4):
        for b in range(4):
            prod = gs[a] * orbs[b]
            for s in range(4):
                base = ((a * 4 + b) * 4 + s) * NC
                term = wgtb[:, base:base + NC] * prod
                acc[s] = term if acc[s] is None else acc[s] + term
    wo = jnp.concatenate(acc, axis=1)
    co_ref[...] = dot(_seg_matrix(), wo) * F32(1.0 / M)


def _density_add(co):
    sq = co * co
    return jnp.concatenate(
        [sq[:, 0:NC], sq[:, NC:2 * NC] + sq[:, 2 * NC:3 * NC] + sq[:, 3 * NC:]],
        axis=1)


def _atom_kernel(co, dprev, w1, b1, w2, b2, w3, b3, dens_ref, coeff_ref, *, scale):
    dot = lambda a, b: jnp.dot(a, b, preferred_element_type=F32)
    dens = (dprev[...] + _density_add(co[...])) * F32(scale)
    dens_ref[...] = dens
    h = _silu(dot(dens, w1[...]) + b1[...])
    h = _silu(dot(h, w2[...]) + b2[...])
    coeff_ref[...] = dot(h, w3[...]) + b3[...]


def _final_kernel(co, dprev, w1, b1, w2, b2, w3, b3, out_ref):
    dot = lambda a, b: jnp.dot(a, b, preferred_element_type=F32)
    dens = (dprev[...] + _density_add(co[...])) * F32(1.0 / math.sqrt(2.0))
    h = _silu(dot(dens, w1[...]) + b1[...])
    h = _silu(dot(h, w2[...]) + b2[...])
    y = dot(h, w3[...]) + b3[...]
    out_ref[...] = jnp.sum(y, axis=0, keepdims=True)


def _full(shape):
    nd = len(shape)
    return pl.BlockSpec(shape, lambda i: (0,) * nd)


def kernel(cart, input_embed, contracted_coeff, ens_cg, p_neigh, p_cg, p_mp, p_out,
           neighlist, index_l, index_i1, index_i2, index_add, index_cg):
    f32 = lambda a: a.astype(F32)
    ids = neighlist.T.astype(jnp.int32)
    cc = f32(contracted_coeff).reshape(3, 2 * NC, NW).transpose(0, 2, 1)

    def mlp_args(p):
        w1, b1, w2, b2, w3, b3 = p
        return [f32(w1), f32(b1).reshape(1, -1), f32(w2), f32(b2).reshape(1, -1),
                f32(w3), f32(b3).reshape(1, -1)]

    oh1 = jax.nn.one_hot(index_i1, 4, dtype=F32)
    oh2 = jax.nn.one_hot(index_i2, 4, dtype=F32)
    ohadd = jax.nn.one_hot(index_add, 4, dtype=F32)
    wmat = jnp.einsum('c,ca,cb,cs->cabs', f32(ens_cg), oh1, oh2, ohadd).reshape(12, 64)
    wmatw = jnp.kron(wmat, jnp.ones((1, NC), dtype=F32))
    sphw = jnp.kron(jnp.eye(4, dtype=F32), jnp.ones((1, NC), dtype=F32))

    pair_specs = (
        [pl.BlockSpec((PB, 2), lambda i: (i, 0)),
         pl.BlockSpec((PB, 16), lambda i: (i, 0)),
         _full((N, 3)), _full((NW, 64))]
        + [_full(s.shape) for s in mlp_args(p_neigh)]
        + [_full(s.shape) for s in mlp_args(p_cg)]
    )
    co0, meta = pl.pallas_call(
        _pair_kernel,
        grid=(NBLK,),
        in_specs=pair_specs,
        out_specs=[pl.BlockSpec((CB, 128), lambda i: (i, 0)),
                   pl.BlockSpec((PB, 36), lambda i: (i, 0))],
        out_shape=[jax.ShapeDtypeStruct((N, 128), F32),
                   jax.ShapeDtypeStruct((P, 36), F32)],
        interpret=_INTERP,
    )(ids, f32(input_embed), f32(cart), cc[0],
      *mlp_args(p_neigh), *mlp_args(p_cg))

    def atom_step(co, dprev, params, scale):
        fn = lambda *refs: _atom_kernel(*refs, scale=scale)
        return pl.pallas_call(
            fn,
            grid=(1,),
            in_specs=[_full((N, 128)), _full((N, 64))] + [_full(s.shape) for s in mlp_args(params)],
            out_specs=[_full((N, 64)), _full((N, 64))],
            out_shape=[jax.ShapeDtypeStruct((N, 64), F32),
                       jax.ShapeDtypeStruct((N, 64), F32)],
            interpret=_INTERP,
        )(co, dprev, *mlp_args(params))

    def inter_step(it, co_tab, cf_tab):
        fn = lambda *refs: _inter_kernel(*refs, it=it)
        return pl.pallas_call(
            fn,
            grid=(NBLK,),
            in_specs=[pl.BlockSpec((PB, 2), lambda i: (i, 0)),
                      pl.BlockSpec((PB, 36), lambda i: (i, 0)),
                      _full((NW, 64)), _full((12, 2048)), _full((4, 128)),
                      _full((N, 128)), _full((N, 64))],
            out_specs=pl.BlockSpec((CB, 128), lambda i: (i, 0)),
            out_shape=jax.ShapeDtypeStruct((N, 128), F32),
            interpret=_INTERP,
        )(ids, meta, cc[it + 1], wmatw, sphw, co_tab, cf_tab)

    zeros64 = jnp.zeros((N, 64), dtype=F32)
    dens0, coeff0 = atom_step(co0, zeros64, p_mp[0], 1.0)
    co1 = inter_step(0, co0, coeff0)
    dens1, coeff1 = atom_step(co1, dens0, p_mp[1], 1.0 / math.sqrt(2.0))
    co2 = inter_step(1, co1, coeff1)

    out = pl.pallas_call(
        _final_kernel,
        grid=(1,),
        in_specs=[_full((N, 128)), _full((N, 64))] + [_full(s.shape) for s in mlp_args(p_out)],
        out_specs=_full((1, 1)),
        out_shape=jax.ShapeDtypeStruct((1, 1), F32),
        interpret=_INTERP,
    )(co2, dens1, *mlp_args(p_out))
    return out[0, 0]


# speedup vs baseline: 4.9356x; 1.6335x over previous
import math

import jax
import jax.numpy as jnp
from jax import lax
from jax.experimental import pallas as pl

N = 192
M = N - 1
P = N * M
NW = 8
NC = 32
CB = 8
PB = CB * M
NBLK = N // CB
C0 = 0.28209479177387814
C1 = 0.4886025119029199
F32 = jnp.float32
_INTERP = False


def _silu(x):
    return x * jax.nn.sigmoid(x)


def _seg_matrix():
    ia = lax.broadcasted_iota(jnp.int32, (CB, PB), 0)
    ir = lax.broadcasted_iota(jnp.int32, (CB, PB), 1)
    lo = ia * M
    return jnp.where((ir >= lo) & (ir < lo + M), F32(1.0), F32(0.0))


def _onehot_pairs(idx_col):
    it = lax.broadcasted_iota(jnp.int32, (PB, N), 1)
    return jnp.where(idx_col == it, F32(1.0), F32(0.0))


def _pair_kernel(ids, emb, cart, cc0,
                 nw1, nb1, nw2, nb2, nw3, nb3,
                 cw1, cb1, cw2, cb2, cw3, cb3,
                 co_ref, meta_ref):
    dot = lambda a, b: jnp.dot(a, b, preferred_element_type=F32)
    x = emb[...]
    h = _silu(dot(x, nw1[...]) + nb1[...])
    h = _silu(dot(h, nw2[...]) + nb2[...])
    e = dot(h, nw3[...]) + nb3[...]

    idsv = ids[...]
    ohc = _onehot_pairs(idsv[:, 0:1])
    ohn = _onehot_pairs(idsv[:, 1:2])
    cartv = cart[...]
    dv = dot(ohn, cartv) - dot(ohc, cartv)
    d2 = jnp.sum(dv * dv, axis=1, keepdims=True)
    dist = jnp.sqrt(d2)

    w = e[:, 0:NW]
    cen = e[:, NW:2 * NW]
    t = w * (dist - cen)
    rad8 = jnp.exp(-(t * t))
    r0 = dot(rad8, cc0[...])

    u = dv / dist
    s1v = C1 * u[:, 1:2]
    s2v = C1 * u[:, 2:3]
    s3v = C1 * u[:, 0:1]

    wr0 = e[:, 2 * NW:] * r0
    lo, hi = wr0[:, 0:NC], wr0[:, NC:]
    worb = jnp.concatenate(
        [lo * C0, hi * s1v, hi * s2v, hi * s3v], axis=1)

    h2 = _silu(dot(x, cw1[...]) + cb1[...])
    h2 = _silu(dot(h2, cw2[...]) + cb2[...])
    cg = dot(h2, cw3[...]) + cb3[...]

    meta_ref[...] = jnp.concatenate(
        [rad8, jnp.full((PB, 1), C0, dtype=F32), s1v, s2v, s3v, cg], axis=1)
    co_ref[...] = dot(_seg_matrix(), worb) * F32(1.0 / M)


BF16 = jnp.bfloat16


def _split(x):
    hi = x.astype(BF16)
    lo = (x - hi.astype(F32)).astype(BF16)
    return hi, lo


def _dot2(x, m_bf):
    hi, lo = _split(x)
    d = lambda a: jnp.dot(a, m_bf, preferred_element_type=F32)
    return d(hi) + d(lo)


def _inter_kernel(ids, meta, cc, sphw, tab, q1, q2, rsel, ens384, aadd,
                  co_ref, *, it):
    dot = lambda a, b: jnp.dot(a, b, preferred_element_type=F32)
    m = meta[...]
    rad8 = m[:, 0:NW]
    sph4 = m[:, NW:NW + 4]
    cg = m[:, 12 + 12 * it:24 + 12 * it]

    ohn = _onehot_pairs(ids[...][:, 1:2]).astype(BF16)
    th, tl = _split(tab[...])
    g2 = dot(ohn, th) + dot(ohn, tl)
    g = g2[:, 0:128]
    cf = g2[:, 128:192]

    r = dot(rad8, cc[...])
    wr = r * cf
    hi = wr[:, NC:]
    orb = jnp.concatenate([wr[:, 0:NC], hi, hi, hi], axis=1) * dot(sph4, sphw[...])

    io1 = _dot2(g, q1[...])
    io2 = _dot2(orb, q2[...])
    cge = dot(cg, rsel[...]) * ens384[...]
    inter = io1 * io2 * cge
    wo = _dot2(inter, aadd[...])
    co_ref[...] = dot(_seg_matrix(), wo) * F32(1.0 / M)


def _density_add(co):
    sq = co * co
    return jnp.concatenate(
        [sq[:, 0:NC], sq[:, NC:2 * NC] + sq[:, 2 * NC:3 * NC] + sq[:, 3 * NC:]],
        axis=1)


def _atom_kernel(co, dprev, w1, b1, w2, b2, w3, b3, dens_ref, coeff_ref, *, scale):
    dot = lambda a, b: jnp.dot(a, b, preferred_element_type=F32)
    dens = (dprev[...] + _density_add(co[...])) * F32(scale)
    dens_ref[...] = dens
    h = _silu(dot(dens, w1[...]) + b1[...])
    h = _silu(dot(h, w2[...]) + b2[...])
    coeff_ref[...] = dot(h, w3[...]) + b3[...]


def _final_kernel(co, dprev, w1, b1, w2, b2, w3, b3, out_ref):
    dot = lambda a, b: jnp.dot(a, b, preferred_element_type=F32)
    dens = (dprev[...] + _density_add(co[...])) * F32(1.0 / math.sqrt(2.0))
    h = _silu(dot(dens, w1[...]) + b1[...])
    h = _silu(dot(h, w2[...]) + b2[...])
    y = dot(h, w3[...]) + b3[...]
    out_ref[...] = jnp.sum(y, axis=0, keepdims=True)


def _full(shape):
    nd = len(shape)
    return pl.BlockSpec(shape, lambda i: (0,) * nd)


def kernel(cart, input_embed, contracted_coeff, ens_cg, p_neigh, p_cg, p_mp, p_out,
           neighlist, index_l, index_i1, index_i2, index_add, index_cg):
    f32 = lambda a: a.astype(F32)
    ids = neighlist.T.astype(jnp.int32)
    cc = f32(contracted_coeff).reshape(3, 2 * NC, NW).transpose(0, 2, 1)

    def mlp_args(p):
        w1, b1, w2, b2, w3, b3 = p
        return [f32(w1), f32(b1).reshape(1, -1), f32(w2), f32(b2).reshape(1, -1),
                f32(w3), f32(b3).reshape(1, -1)]

    oh1 = jax.nn.one_hot(index_i1, 4, dtype=F32)
    oh2 = jax.nn.one_hot(index_i2, 4, dtype=F32)
    ohadd = jax.nn.one_hot(index_add, 4, dtype=F32)
    sphw = jnp.kron(jnp.eye(4, dtype=F32), jnp.ones((1, NC), dtype=F32))
    eye32 = jnp.eye(NC, dtype=F32)
    q1 = jnp.kron(oh1.T, eye32).astype(jnp.bfloat16)
    q2 = jnp.kron(oh2.T, eye32).astype(jnp.bfloat16)
    aadd = jnp.kron(ohadd, eye32).astype(jnp.bfloat16)
    rsel = jnp.kron(jnp.eye(12, dtype=F32), jnp.ones((1, NC), dtype=F32))
    ens384 = jnp.kron(f32(ens_cg), jnp.ones((NC,), dtype=F32)).reshape(1, 384)

    pair_specs = (
        [pl.BlockSpec((PB, 2), lambda i: (i, 0)),
         pl.BlockSpec((PB, 16), lambda i: (i, 0)),
         _full((N, 3)), _full((NW, 64))]
        + [_full(s.shape) for s in mlp_args(p_neigh)]
        + [_full(s.shape) for s in mlp_args(p_cg)]
    )
    co0, meta = pl.pallas_call(
        _pair_kernel,
        grid=(NBLK,),
        in_specs=pair_specs,
        out_specs=[pl.BlockSpec((CB, 128), lambda i: (i, 0)),
                   pl.BlockSpec((PB, 36), lambda i: (i, 0))],
        out_shape=[jax.ShapeDtypeStruct((N, 128), F32),
                   jax.ShapeDtypeStruct((P, 36), F32)],
        interpret=_INTERP,
    )(ids, f32(input_embed), f32(cart), cc[0],
      *mlp_args(p_neigh), *mlp_args(p_cg))

    def atom_step(co, dprev, params, scale):
        fn = lambda *refs: _atom_kernel(*refs, scale=scale)
        return pl.pallas_call(
            fn,
            grid=(1,),
            in_specs=[_full((N, 128)), _full((N, 64))] + [_full(s.shape) for s in mlp_args(params)],
            out_specs=[_full((N, 64)), _full((N, 64))],
            out_shape=[jax.ShapeDtypeStruct((N, 64), F32),
                       jax.ShapeDtypeStruct((N, 64), F32)],
            interpret=_INTERP,
        )(co, dprev, *mlp_args(params))

    def inter_step(it, co_tab, cf_tab):
        fn = lambda *refs: _inter_kernel(*refs, it=it)
        tab = jnp.concatenate([co_tab, cf_tab], axis=1)
        return pl.pallas_call(
            fn,
            grid=(NBLK,),
            in_specs=[pl.BlockSpec((PB, 2), lambda i: (i, 0)),
                      pl.BlockSpec((PB, 36), lambda i: (i, 0)),
                      _full((NW, 64)), _full((4, 128)), _full((N, 192)),
                      _full((128, 384)), _full((128, 384)),
                      _full((12, 384)), _full((1, 384)), _full((384, 128))],
            out_specs=pl.BlockSpec((CB, 128), lambda i: (i, 0)),
            out_shape=jax.ShapeDtypeStruct((N, 128), F32),
            interpret=_INTERP,
        )(ids, meta, cc[it + 1], sphw, tab, q1, q2, rsel, ens384, aadd)

    zeros64 = jnp.zeros((N, 64), dtype=F32)
    dens0, coeff0 = atom_step(co0, zeros64, p_mp[0], 1.0)
    co1 = inter_step(0, co0, coeff0)
    dens1, coeff1 = atom_step(co1, dens0, p_mp[1], 1.0 / math.sqrt(2.0))
    co2 = inter_step(1, co1, coeff1)

    out = pl.pallas_call(
        _final_kernel,
        grid=(1,),
        in_specs=[_full((N, 128)), _full((N, 64))] + [_full(s.shape) for s in mlp_args(p_out)],
        out_specs=_full((1, 1)),
        out_shape=jax.ShapeDtypeStruct((1, 1), F32),
        interpret=_INTERP,
    )(co2, dens1, *mlp_args(p_out))
    return out[0, 0]


# speedup vs baseline: 5.5758x; 1.1297x over previous
import math

import jax
import jax.numpy as jnp
from jax import lax
from jax.experimental import pallas as pl

N = 192
M = N - 1
P = N * M
NW = 8
NC = 32
CB = 8
PB = CB * M
NBLK = N // CB
C0 = 0.28209479177387814
C1 = 0.4886025119029199
F32 = jnp.float32
_INTERP = False


def _silu(x):
    return x * jax.nn.sigmoid(x)


def _seg_matrix():
    ia = lax.broadcasted_iota(jnp.int32, (CB, PB), 0)
    ir = lax.broadcasted_iota(jnp.int32, (CB, PB), 1)
    lo = ia * M
    return jnp.where((ir >= lo) & (ir < lo + M), F32(1.0), F32(0.0))


def _onehot_pairs(idx_col):
    it = lax.broadcasted_iota(jnp.int32, (PB, N), 1)
    return jnp.where(idx_col == it, F32(1.0), F32(0.0))


def _pair_kernel(ids, emb, cart, cc0,
                 nw1, nb1, nw2, nb2, nw3, nb3,
                 cw1, cb1, cw2, cb2, cw3, cb3,
                 co_ref, meta_ref):
    dot = lambda a, b: jnp.dot(a, b, preferred_element_type=F32)
    x = emb[...]
    h = _silu(dot(x, nw1[...]) + nb1[...])
    h = _silu(dot(h, nw2[...]) + nb2[...])
    e = dot(h, nw3[...]) + nb3[...]

    idsv = ids[...]
    ohc = _onehot_pairs(idsv[:, 0:1])
    ohn = _onehot_pairs(idsv[:, 1:2])
    cartv = cart[...]
    dv = dot(ohn, cartv) - dot(ohc, cartv)
    d2 = jnp.sum(dv * dv, axis=1, keepdims=True)
    dist = jnp.sqrt(d2)

    w = e[:, 0:NW]
    cen = e[:, NW:2 * NW]
    t = w * (dist - cen)
    rad8 = jnp.exp(-(t * t))
    r0 = dot(rad8, cc0[...])

    u = dv / dist
    s1v = C1 * u[:, 1:2]
    s2v = C1 * u[:, 2:3]
    s3v = C1 * u[:, 0:1]

    wr0 = e[:, 2 * NW:] * r0
    lo, hi = wr0[:, 0:NC], wr0[:, NC:]
    worb = jnp.concatenate(
        [lo * C0, hi * s1v, hi * s2v, hi * s3v], axis=1)

    h2 = _silu(dot(x, cw1[...]) + cb1[...])
    h2 = _silu(dot(h2, cw2[...]) + cb2[...])
    cg = dot(h2, cw3[...]) + cb3[...]

    meta_ref[...] = jnp.concatenate(
        [rad8, jnp.full((PB, 1), C0, dtype=F32), s1v, s2v, s3v, cg], axis=1)
    co_ref[...] = dot(_seg_matrix(), worb) * F32(1.0 / M)


BF16 = jnp.bfloat16


def _split(x):
    hi = x.astype(BF16)
    lo = (x - hi.astype(F32)).astype(BF16)
    return hi, lo


def _dot2(x, m_bf):
    hi, lo = _split(x)
    d = lambda a: jnp.dot(a, m_bf, preferred_element_type=F32)
    return d(hi) + d(lo)


def _inter_kernel(ids, meta, cc, sphw, tab, q1, q2, rsel, ens384, aadd,
                  co_ref, *, it):
    dot = lambda a, b: jnp.dot(a, b, preferred_element_type=F32)
    m = meta[...]
    rad8 = m[:, 0:NW]
    sph4 = m[:, NW:NW + 4]
    cg = m[:, 12 + 12 * it:24 + 12 * it]

    ohn = _onehot_pairs(ids[...][:, 1:2])
    g2 = dot(ohn, tab[...])
    g = g2[:, 0:128]
    cf = g2[:, 128:192]

    r = dot(rad8, cc[...])
    wr = r * cf
    hi = wr[:, NC:]
    orb = jnp.concatenate([wr[:, 0:NC], hi, hi, hi], axis=1) * dot(sph4, sphw[...])

    io1 = dot(g, q1[...])
    io2 = dot(orb, q2[...])
    cge = dot(cg, rsel[...]) * ens384[...]
    inter = io1 * io2 * cge
    wo = dot(inter, aadd[...])
    co_ref[...] = dot(_seg_matrix(), wo) * F32(1.0 / M)


def _density_add(co):
    sq = co * co
    return jnp.concatenate(
        [sq[:, 0:NC], sq[:, NC:2 * NC] + sq[:, 2 * NC:3 * NC] + sq[:, 3 * NC:]],
        axis=1)


def _atom_kernel(co, dprev, w1, b1, w2, b2, w3, b3, dens_ref, coeff_ref, *, scale):
    dot = lambda a, b: jnp.dot(a, b, preferred_element_type=F32)
    dens = (dprev[...] + _density_add(co[...])) * F32(scale)
    dens_ref[...] = dens
    h = _silu(dot(dens, w1[...]) + b1[...])
    h = _silu(dot(h, w2[...]) + b2[...])
    coeff_ref[...] = dot(h, w3[...]) + b3[...]


def _final_kernel(co, dprev, w1, b1, w2, b2, w3, b3, out_ref):
    dot = lambda a, b: jnp.dot(a, b, preferred_element_type=F32)
    dens = (dprev[...] + _density_add(co[...])) * F32(1.0 / math.sqrt(2.0))
    h = _silu(dot(dens, w1[...]) + b1[...])
    h = _silu(dot(h, w2[...]) + b2[...])
    y = dot(h, w3[...]) + b3[...]
    out_ref[...] = jnp.sum(y, axis=0, keepdims=True)


def _full(shape):
    nd = len(shape)
    return pl.BlockSpec(shape, lambda i: (0,) * nd)


def kernel(cart, input_embed, contracted_coeff, ens_cg, p_neigh, p_cg, p_mp, p_out,
           neighlist, index_l, index_i1, index_i2, index_add, index_cg):
    f32 = lambda a: a.astype(F32)
    ids = neighlist.T.astype(jnp.int32)
    cc = f32(contracted_coeff).reshape(3, 2 * NC, NW).transpose(0, 2, 1)

    def mlp_args(p):
        w1, b1, w2, b2, w3, b3 = p
        return [f32(w1), f32(b1).reshape(1, -1), f32(w2), f32(b2).reshape(1, -1),
                f32(w3), f32(b3).reshape(1, -1)]

    oh1 = jax.nn.one_hot(index_i1, 4, dtype=F32)
    oh2 = jax.nn.one_hot(index_i2, 4, dtype=F32)
    ohadd = jax.nn.one_hot(index_add, 4, dtype=F32)
    sphw = jnp.kron(jnp.eye(4, dtype=F32), jnp.ones((1, NC), dtype=F32))
    eye32 = jnp.eye(NC, dtype=F32)
    q1 = jnp.kron(oh1.T, eye32)
    q2 = jnp.kron(oh2.T, eye32)
    aadd = jnp.kron(ohadd, eye32)
    rsel = jnp.kron(jnp.eye(12, dtype=F32), jnp.ones((1, NC), dtype=F32))
    ens384 = jnp.kron(f32(ens_cg), jnp.ones((NC,), dtype=F32)).reshape(1, 384)

    pair_specs = (
        [pl.BlockSpec((PB, 2), lambda i: (i, 0)),
         pl.BlockSpec((PB, 16), lambda i: (i, 0)),
         _full((N, 3)), _full((NW, 64))]
        + [_full(s.shape) for s in mlp_args(p_neigh)]
        + [_full(s.shape) for s in mlp_args(p_cg)]
    )
    co0, meta = pl.pallas_call(
        _pair_kernel,
        grid=(NBLK,),
        in_specs=pair_specs,
        out_specs=[pl.BlockSpec((CB, 128), lambda i: (i, 0)),
                   pl.BlockSpec((PB, 36), lambda i: (i, 0))],
        out_shape=[jax.ShapeDtypeStruct((N, 128), F32),
                   jax.ShapeDtypeStruct((P, 36), F32)],
        interpret=_INTERP,
    )(ids, f32(input_embed), f32(cart), cc[0],
      *mlp_args(p_neigh), *mlp_args(p_cg))

    def atom_step(co, dprev, params, scale):
        fn = lambda *refs: _atom_kernel(*refs, scale=scale)
        return pl.pallas_call(
            fn,
            grid=(1,),
            in_specs=[_full((N, 128)), _full((N, 64))] + [_full(s.shape) for s in mlp_args(params)],
            out_specs=[_full((N, 64)), _full((N, 64))],
            out_shape=[jax.ShapeDtypeStruct((N, 64), F32),
                       jax.ShapeDtypeStruct((N, 64), F32)],
            interpret=_INTERP,
        )(co, dprev, *mlp_args(params))

    def inter_step(it, co_tab, cf_tab):
        fn = lambda *refs: _inter_kernel(*refs, it=it)
        tab = jnp.concatenate([co_tab, cf_tab], axis=1)
        return pl.pallas_call(
            fn,
            grid=(NBLK,),
            in_specs=[pl.BlockSpec((PB, 2), lambda i: (i, 0)),
                      pl.BlockSpec((PB, 36), lambda i: (i, 0)),
                      _full((NW, 64)), _full((4, 128)), _full((N, 192)),
                      _full((128, 384)), _full((128, 384)),
                      _full((12, 384)), _full((1, 384)), _full((384, 128))],
            out_specs=pl.BlockSpec((CB, 128), lambda i: (i, 0)),
            out_shape=jax.ShapeDtypeStruct((N, 128), F32),
            interpret=_INTERP,
        )(ids, meta, cc[it + 1], sphw, tab, q1, q2, rsel, ens384, aadd)

    zeros64 = jnp.zeros((N, 64), dtype=F32)
    dens0, coeff0 = atom_step(co0, zeros64, p_mp[0], 1.0)
    co1 = inter_step(0, co0, coeff0)
    dens1, coeff1 = atom_step(co1, dens0, p_mp[1], 1.0 / math.sqrt(2.0))
    co2 = inter_step(1, co1, coeff1)

    out = pl.pallas_call(
        _final_kernel,
        grid=(1,),
        in_specs=[_full((N, 128)), _full((N, 64))] + [_full(s.shape) for s in mlp_args(p_out)],
        out_specs=_full((1, 1)),
        out_shape=jax.ShapeDtypeStruct((1, 1), F32),
        interpret=_INTERP,
    )(co2, dens1, *mlp_args(p_out))
    return out[0, 0]
